# Initial kernel scaffold; baseline (speedup 1.0000x reference)
import jax
import jax.numpy as jnp
from jax.experimental import pallas as pl


def kernel(x_v, x_e, edge_orders, indices_with_nodes, params):
    raise NotImplementedError("write your pallas kernel here")


# speedup vs baseline: 76.6585x; 76.6585x over previous
import functools
import math

import jax
import jax.numpy as jnp
import numpy as np
from jax import lax
from jax.experimental import pallas as pl
from jax.experimental.pallas import tpu as pltpu
from jax.experimental.pallas import tpu_sc as plsc

N = 10000
E = 5000
M = 320000
EN = E + N
D = 128
H = 8
DH = 16
HID = 256
PE_DIM = 64
MAXK = 16

NC = 2
NS = 16
SC_B = 80
PER_SC = M // NC
PER_TILE = PER_SC // NS
NBLK = PER_TILE // SC_B
N_PAD = 10240
ZROWS = N_PAD // NS


def _sin_pe(max_len, d):
    pos = np.arange(max_len)[:, None].astype(np.float64)
    div = np.exp(np.arange(0, d, 2).astype(np.float64) * (-math.log(10000.0) / d))
    pe = np.zeros((max_len, d), dtype=np.float64)
    pe[:, 0::2] = np.sin(pos * div)
    pe[:, 1::2] = np.cos(pos * div)
    return pe.astype(np.float32)


_PE1 = np.zeros((32, D), np.float32)
_PE1[: MAXK + 1] = _sin_pe(MAXK + 1, D)
_PE2 = _sin_pe(2, D)
_PEQ = _sin_pe(2, PE_DIM)

_PRE_BLK = 600
_PRE_GRID = EN // _PRE_BLK
_POST_BLK = 1000
_POST_GRID = N // _POST_BLK


def _ln(x, g, b):
    mu = jnp.mean(x, axis=-1, keepdims=True)
    var = jnp.mean((x - mu) ** 2, axis=-1, keepdims=True)
    return (x - mu) * lax.rsqrt(var + 1e-5) * g + b


def _head_expand_mat(dtype=jnp.float32):
    r = lax.broadcasted_iota(jnp.int32, (16, D), 0)
    c = lax.broadcasted_iota(jnp.int32, (16, D), 1) // DH
    return (r == c).astype(dtype)


def _pre_body(x_ref, oh_ref, ln1g, ln1b, pe1t, w1a, w1b, b1, w2, b2,
              kw, kb, vw, vb, qw1, qb1, qw2, qb2, peq,
              v_ref, lg_ref, att0_ref, gmax_ref,
              macc, sacc, aacc, gmacc):
    i = pl.program_id(0)

    xb = x_ref[...]
    ln = _ln(xb, ln1g[...], ln1b[...])
    pe = oh_ref[...] @ pe1t[...]
    h1 = jnp.maximum(ln @ w1a[...] + pe @ w1b[...] + b1[...], 0.0)
    xh = xb + h1 @ w2[...] + b2[...]
    k = xh @ kw[...] + kb[...]
    vv = xh @ vw[...] + vb[...]
    v_ref[...] = vv

    qh = jnp.maximum(peq[...] @ qw1[...] + qb1[...], 0.0) @ qw2[...] + qb2[...]
    q0f = qh[0:1, :]
    q1f = qh[1:2, :]

    sr = lax.broadcasted_iota(jnp.int32, (D, 16), 0) // DH
    sc = lax.broadcasted_iota(jnp.int32, (D, 16), 1)
    s16 = (sr == sc).astype(jnp.float32)
    st = _head_expand_mat()

    l0s = ((k[:, :D] * q0f) @ s16) * 0.25
    l1s = ((k[:, D:] * q1f) @ s16) * 0.25
    lg_ref[...] = l1s
    l0e = l0s @ st
    l1e = l1s @ st

    @pl.when(i == 0)
    def _init():
        macc[...] = jnp.full((1, D), -1e30, jnp.float32)
        sacc[...] = jnp.zeros((1, D), jnp.float32)
        aacc[...] = jnp.zeros((1, D), jnp.float32)
        gmacc[...] = jnp.full((1, D), -1e30, jnp.float32)

    m_old = macc[...]
    m_new = jnp.maximum(m_old, jnp.max(l0e, axis=0, keepdims=True))
    corr = jnp.exp(m_old - m_new)
    p = jnp.exp(l0e - m_new)
    sacc[...] = sacc[...] * corr + jnp.sum(p, axis=0, keepdims=True)
    aacc[...] = aacc[...] * corr + jnp.sum(p * vv, axis=0, keepdims=True)
    macc[...] = m_new
    gmacc[...] = jnp.maximum(gmacc[...], jnp.max(l1e, axis=0, keepdims=True))

    @pl.when(i == _PRE_GRID - 1)
    def _fin():
        att0_ref[...] = aacc[...] / sacc[...]
        gmax_ref[...] = gmacc[...]


def _pre_call(x, oh, p):
    cb = lambda shape: pl.BlockSpec(shape, lambda i: (0, 0))
    out_shapes = (
        jax.ShapeDtypeStruct((EN, D), jnp.float32),
        jax.ShapeDtypeStruct((EN, 16), jnp.float32),
        jax.ShapeDtypeStruct((1, D), jnp.float32),
        jax.ShapeDtypeStruct((1, D), jnp.float32),
    )
    return pl.pallas_call(
        _pre_body,
        grid=(_PRE_GRID,),
        in_specs=[
            pl.BlockSpec((_PRE_BLK, D), lambda i: (i, 0)),
            pl.BlockSpec((_PRE_BLK, 32), lambda i: (i, 0)),
            cb((1, D)), cb((1, D)), cb((32, D)),
            cb((D, HID)), cb((D, HID)), cb((1, HID)), cb((HID, D)), cb((1, D)),
            cb((D, 2 * D)), cb((1, 2 * D)), cb((D, D)), cb((1, D)),
            cb((PE_DIM, HID)), cb((1, HID)), cb((HID, H * DH)), cb((1, H * DH)),
            cb((2, PE_DIM)),
        ],
        out_specs=(
            pl.BlockSpec((_PRE_BLK, D), lambda i: (i, 0)),
            pl.BlockSpec((_PRE_BLK, 16), lambda i: (i, 0)),
            pl.BlockSpec((1, D), lambda i: (0, 0)),
            pl.BlockSpec((1, D), lambda i: (0, 0)),
        ),
        out_shape=out_shapes,
        scratch_shapes=[pltpu.VMEM((1, D), jnp.float32)] * 4,
    )(
        x, oh,
        p['ln1g'].reshape(1, D), p['ln1b'].reshape(1, D), jnp.asarray(_PE1),
        p['m1W1'][:D], p['m1W1'][D:], p['m1b1'].reshape(1, HID),
        p['m1W2'], p['m1b2'].reshape(1, D),
        p['kW'], p['kb'].reshape(1, 2 * D), p['vW'], p['vb'].reshape(1, D),
        p['qW1'], p['qb1'].reshape(1, HID), p['qW2'], p['qb2'].reshape(1, H * DH),
        jnp.asarray(_PEQ),
    )


def _sc_body(idx0_hbm, idx1_hbm, lg_hbm, v_hbm, gmax_hbm, zn_hbm, zd_hbm,
             num_hbm, den_hbm,
             idx0_v, idx1_v, lrows, vrows, wrows, exd, gmax_v,
             num_sh, den_sh, sem1, sem2):
    c = lax.axis_index("c")
    s = lax.axis_index("s")

    pltpu.sync_copy(zn_hbm.at[pl.ds(s * ZROWS, ZROWS)],
                    num_sh.at[pl.ds(s * ZROWS, ZROWS)])
    pltpu.sync_copy(zd_hbm.at[pl.ds(s * ZROWS, ZROWS)],
                    den_sh.at[pl.ds(s * ZROWS, ZROWS)])
    pltpu.sync_copy(gmax_hbm, gmax_v)
    plsc.subcore_barrier()

    gm = gmax_v[...]
    base = c * PER_SC + s * PER_TILE

    def blk(i, carry):
        off = base + i * SC_B
        pltpu.sync_copy(idx0_hbm.at[pl.ds(off, SC_B)], idx0_v)
        pltpu.sync_copy(idx1_hbm.at[pl.ds(off, SC_B)], idx1_v)
        cp1 = pltpu.async_copy(lg_hbm.at[idx1_v], lrows, sem1)
        cp2 = pltpu.async_copy(v_hbm.at[idx1_v], vrows, sem2)
        cp1.wait()
        cp2.wait()
        for r in range(SC_B):
            ex = jnp.exp(lrows[r, :] - gm)
            exd[r, :] = ex
            for h in range(H):
                w = jnp.full((16,), ex[h], jnp.float32)
                wrows[r, pl.ds(h * DH, DH)] = w * vrows[r, pl.ds(h * DH, DH)]
        pltpu.sync_copy(exd, den_sh.at[idx0_v], add=True)
        pltpu.sync_copy(wrows, num_sh.at[idx0_v], add=True)
        return carry

    lax.fori_loop(0, NBLK, blk, 0)
    plsc.subcore_barrier()

    r0 = s * ZROWS
    pltpu.sync_copy(num_sh.at[pl.ds(r0, ZROWS)], num_hbm.at[c, pl.ds(r0, ZROWS)])
    pltpu.sync_copy(den_sh.at[pl.ds(r0, ZROWS)], den_hbm.at[c, pl.ds(r0, ZROWS)])


@functools.partial(jax.jit, static_argnames=())
def _sc_call(idx0, idx1, lg, v, gmax16, zn, zd):
    mesh = plsc.VectorSubcoreMesh(core_axis_name="c", subcore_axis_name="s",
                                  num_cores=NC, num_subcores=NS)
    kfn = pl.kernel(
        _sc_body,
        out_type=(
            jax.ShapeDtypeStruct((NC, N_PAD, D), jnp.float32),
            jax.ShapeDtypeStruct((NC, N_PAD, 16), jnp.float32),
        ),
        mesh=mesh,
        scratch_types=[
            pltpu.VMEM((SC_B,), jnp.int32),
            pltpu.VMEM((SC_B,), jnp.int32),
            pltpu.VMEM((SC_B, 16), jnp.float32),
            pltpu.VMEM((SC_B, D), jnp.float32),
            pltpu.VMEM((SC_B, D), jnp.float32),
            pltpu.VMEM((SC_B, 16), jnp.float32),
            pltpu.VMEM((16,), jnp.float32),
            pltpu.VMEM_SHARED((N_PAD, D), jnp.float32),
            pltpu.VMEM_SHARED((N_PAD, 16), jnp.float32),
            pltpu.SemaphoreType.DMA,
            pltpu.SemaphoreType.DMA,
        ],
        compiler_params=pltpu.CompilerParams(use_tc_tiling_on_sc=False),
    )
    return kfn(idx0, idx1, lg, v, gmax16, zn, zd)


def _post_body(n0_ref, n1_ref, d0_ref, d1_ref, att0_ref,
               ln2g, ln2b, w1a, w1b, b1, w2, b2,
               ln3g, ln3b, w31, b31, w32, b32, bias, pe2,
               out_ref):
    st = _head_expand_mat()
    den = (d0_ref[...] + d1_ref[...]) @ st + 1e-12
    att1 = (n0_ref[...] + n1_ref[...]) / den

    a0 = att0_ref[...]
    h0 = jnp.maximum(_ln(a0, ln2g[...], ln2b[...]) @ w1a[...]
                     + pe2[0:1, :] @ w1b[...] + b1[...], 0.0)
    a0p = a0 + h0 @ w2[...] + b2[...]

    h1 = jnp.maximum(_ln(att1, ln2g[...], ln2b[...]) @ w1a[...]
                     + pe2[1:2, :] @ w1b[...] + b1[...], 0.0)
    a1p = att1 + h1 @ w2[...] + b2[...]

    xx = a0p + a1p
    h3 = jnp.maximum(_ln(xx, ln3g[...], ln3b[...]) @ w31[...] + b31[...], 0.0)
    xx = xx + h3 @ w32[...] + b32[...]
    out_ref[...] = xx + bias[...]


def _post_call(n0, n1, d0, d1, att0, p):
    cb = lambda shape: pl.BlockSpec(shape, lambda i: (0, 0))
    return pl.pallas_call(
        _post_body,
        grid=(_POST_GRID,),
        in_specs=[
            pl.BlockSpec((_POST_BLK, D), lambda i: (i, 0)),
            pl.BlockSpec((_POST_BLK, D), lambda i: (i, 0)),
            pl.BlockSpec((_POST_BLK, 16), lambda i: (i, 0)),
            pl.BlockSpec((_POST_BLK, 16), lambda i: (i, 0)),
            cb((1, D)),
            cb((1, D)), cb((1, D)),
            cb((D, HID)), cb((D, HID)), cb((1, HID)), cb((HID, D)), cb((1, D)),
            cb((1, D)), cb((1, D)),
            cb((D, HID)), cb((1, HID)), cb((HID, D)), cb((1, D)),
            cb((1, D)), cb((2, D)),
        ],
        out_specs=pl.BlockSpec((_POST_BLK, D), lambda i: (i, 0)),
        out_shape=jax.ShapeDtypeStruct((N, D), jnp.float32),
    )(
        n0, n1, d0, d1, att0,
        p['ln2g'].reshape(1, D), p['ln2b'].reshape(1, D),
        p['m2W1'][:D], p['m2W1'][D:], p['m2b1'].reshape(1, HID),
        p['m2W2'], p['m2b2'].reshape(1, D),
        p['ln3g'].reshape(1, D), p['ln3b'].reshape(1, D),
        p['m3W1'], p['m3b1'].reshape(1, HID), p['m3W2'], p['m3b2'].reshape(1, D),
        p['bias'].reshape(1, D), jnp.asarray(_PE2),
    )


def kernel(x_v, x_e, edge_orders, indices_with_nodes, params):
    x = jnp.concatenate([x_e, x_v], axis=0)
    orders = jnp.concatenate(
        [edge_orders.astype(jnp.int32), jnp.ones((N,), jnp.int32)])
    oh = (orders[:, None] == jnp.arange(32, dtype=jnp.int32)[None, :]
          ).astype(jnp.float32)

    v_all, lg, att0, gmax_rep = _pre_call(x, oh, params)

    gmax16 = jnp.concatenate(
        [gmax_rep[0, ::DH], jnp.zeros((8,), jnp.float32)])
    idx0 = indices_with_nodes[0]
    idx1 = indices_with_nodes[1]
    zn = jnp.zeros((N_PAD, D), jnp.float32)
    zd = jnp.zeros((N_PAD, 16), jnp.float32)
    num, den = _sc_call(idx0, idx1, lg, v_all, gmax16, zn, zd)

    return _post_call(num[0, :N], num[1, :N], den[0, :N], den[1, :N],
                      att0, params)


# speedup vs baseline: 106.7067x; 1.3920x over previous
import functools
import math

import jax
import jax.numpy as jnp
import numpy as np
from jax import lax
from jax.experimental import pallas as pl
from jax.experimental.pallas import tpu as pltpu
from jax.experimental.pallas import tpu_sc as plsc

N = 10000
E = 5000
M = 320000
EN = E + N
D = 128
H = 8
DH = 16
HID = 256
PE_DIM = 64
MAXK = 16

NC = 2
NS = 16
SC_B = 80
NCHUNK = M // SC_B
TBLK = NCHUNK // NS
NBUF = 2
NROUND = TBLK // NBUF
TW = 80
VH = 64
N_PAD = 10112
ZROWS = N_PAD // NS


def _sin_pe(max_len, d):
    pos = np.arange(max_len)[:, None].astype(np.float64)
    div = np.exp(np.arange(0, d, 2).astype(np.float64) * (-math.log(10000.0) / d))
    pe = np.zeros((max_len, d), dtype=np.float64)
    pe[:, 0::2] = np.sin(pos * div)
    pe[:, 1::2] = np.cos(pos * div)
    return pe.astype(np.float32)


_PE1 = np.zeros((32, D), np.float32)
_PE1[: MAXK + 1] = _sin_pe(MAXK + 1, D)
_PE2 = _sin_pe(2, D)
_PEQ = _sin_pe(2, PE_DIM)

_PRE_BLK = 600
_PRE_GRID = EN // _PRE_BLK
_POST_BLK = 1000
_POST_GRID = N // _POST_BLK


def _ln(x, g, b):
    mu = jnp.mean(x, axis=-1, keepdims=True)
    var = jnp.mean((x - mu) ** 2, axis=-1, keepdims=True)
    return (x - mu) * lax.rsqrt(var + 1e-5) * g + b


def _head_expand_mat(dtype=jnp.float32):
    r = lax.broadcasted_iota(jnp.int32, (16, D), 0)
    c = lax.broadcasted_iota(jnp.int32, (16, D), 1) // DH
    return (r == c).astype(dtype)


def _pre_body(x_ref, oh_ref, ln1g, ln1b, pe1t, w1a, w1b, b1, w2, b2,
              kw, kb, vw, vb, qw1, qb1, qw2, qb2, peq,
              v_ref, lg_ref, att0_ref, gmax_ref,
              macc, sacc, aacc, gmacc):
    i = pl.program_id(0)

    xb = x_ref[...]
    ln = _ln(xb, ln1g[...], ln1b[...])
    pe = oh_ref[...] @ pe1t[...]
    h1 = jnp.maximum(ln @ w1a[...] + pe @ w1b[...] + b1[...], 0.0)
    xh = xb + h1 @ w2[...] + b2[...]
    k = xh @ kw[...] + kb[...]
    vv = xh @ vw[...] + vb[...]
    v_ref[...] = vv

    qh = jnp.maximum(peq[...] @ qw1[...] + qb1[...], 0.0) @ qw2[...] + qb2[...]
    q0f = qh[0:1, :]
    q1f = qh[1:2, :]

    sr = lax.broadcasted_iota(jnp.int32, (D, 16), 0) // DH
    sc = lax.broadcasted_iota(jnp.int32, (D, 16), 1)
    s16 = (sr == sc).astype(jnp.float32)
    st = _head_expand_mat()

    l0s = ((k[:, :D] * q0f) @ s16) * 0.25
    l1s = ((k[:, D:] * q1f) @ s16) * 0.25
    lg_ref[...] = l1s
    l0e = l0s @ st
    l1e = l1s @ st

    @pl.when(i == 0)
    def _init():
        macc[...] = jnp.full((1, D), -1e30, jnp.float32)
        sacc[...] = jnp.zeros((1, D), jnp.float32)
        aacc[...] = jnp.zeros((1, D), jnp.float32)
        gmacc[...] = jnp.full((1, D), -1e30, jnp.float32)

    m_old = macc[...]
    m_new = jnp.maximum(m_old, jnp.max(l0e, axis=0, keepdims=True))
    corr = jnp.exp(m_old - m_new)
    p = jnp.exp(l0e - m_new)
    sacc[...] = sacc[...] * corr + jnp.sum(p, axis=0, keepdims=True)
    aacc[...] = aacc[...] * corr + jnp.sum(p * vv, axis=0, keepdims=True)
    macc[...] = m_new
    gmacc[...] = jnp.maximum(gmacc[...], jnp.max(l1e, axis=0, keepdims=True))

    @pl.when(i == _PRE_GRID - 1)
    def _fin():
        att0_ref[...] = aacc[...] / sacc[...]
        gmax_ref[...] = gmacc[...]


def _pre_call(x, oh, p):
    cb = lambda shape: pl.BlockSpec(shape, lambda i: (0, 0))
    out_shapes = (
        jax.ShapeDtypeStruct((EN, D), jnp.float32),
        jax.ShapeDtypeStruct((EN, 16), jnp.float32),
        jax.ShapeDtypeStruct((1, D), jnp.float32),
        jax.ShapeDtypeStruct((1, D), jnp.float32),
    )
    return pl.pallas_call(
        _pre_body,
        grid=(_PRE_GRID,),
        in_specs=[
            pl.BlockSpec((_PRE_BLK, D), lambda i: (i, 0)),
            pl.BlockSpec((_PRE_BLK, 32), lambda i: (i, 0)),
            cb((1, D)), cb((1, D)), cb((32, D)),
            cb((D, HID)), cb((D, HID)), cb((1, HID)), cb((HID, D)), cb((1, D)),
            cb((D, 2 * D)), cb((1, 2 * D)), cb((D, D)), cb((1, D)),
            cb((PE_DIM, HID)), cb((1, HID)), cb((HID, H * DH)), cb((1, H * DH)),
            cb((2, PE_DIM)),
        ],
        out_specs=(
            pl.BlockSpec((_PRE_BLK, D), lambda i: (i, 0)),
            pl.BlockSpec((_PRE_BLK, 16), lambda i: (i, 0)),
            pl.BlockSpec((1, D), lambda i: (0, 0)),
            pl.BlockSpec((1, D), lambda i: (0, 0)),
        ),
        out_shape=out_shapes,
        scratch_shapes=[pltpu.VMEM((1, D), jnp.float32)] * 4,
    )(
        x, oh,
        p['ln1g'].reshape(1, D), p['ln1b'].reshape(1, D), jnp.asarray(_PE1),
        p['m1W1'][:D], p['m1W1'][D:], p['m1b1'].reshape(1, HID),
        p['m1W2'], p['m1b2'].reshape(1, D),
        p['kW'], p['kb'].reshape(1, 2 * D), p['vW'], p['vb'].reshape(1, D),
        p['qW1'], p['qb1'].reshape(1, HID), p['qW2'], p['qb2'].reshape(1, H * DH),
        jnp.asarray(_PEQ),
    )


def _mid_body(lg_ref, v_ref, gmax_ref, t0_ref, t1_ref):
    ex = jnp.exp(lg_ref[...] - gmax_ref[...])
    wv = (ex @ _head_expand_mat()) * v_ref[...]
    t0_ref[...] = jnp.concatenate([wv[:, :VH], ex], axis=1)
    t1_ref[...] = jnp.concatenate([wv[:, VH:], ex], axis=1)


def _mid_call(lg, v, gmax_hot):
    return pl.pallas_call(
        _mid_body,
        grid=(_PRE_GRID,),
        in_specs=[
            pl.BlockSpec((_PRE_BLK, 16), lambda i: (i, 0)),
            pl.BlockSpec((_PRE_BLK, D), lambda i: (i, 0)),
            pl.BlockSpec((1, 16), lambda i: (0, 0)),
        ],
        out_specs=(
            pl.BlockSpec((_PRE_BLK, TW), lambda i: (i, 0)),
            pl.BlockSpec((_PRE_BLK, TW), lambda i: (i, 0)),
        ),
        out_shape=(
            jax.ShapeDtypeStruct((EN, TW), jnp.float32),
            jax.ShapeDtypeStruct((EN, TW), jnp.float32),
        ),
    )(lg, v, gmax_hot)


def _sc_body(idx0_hbm, idx1_hbm, tbl_hbm, z_hbm, acc_hbm,
             idx0_st, idx1_st, rows, acc_sh, gsem):
    c = lax.axis_index("c")
    s = lax.axis_index("s")

    pltpu.sync_copy(z_hbm.at[pl.ds(s * ZROWS, ZROWS)],
                    acc_sh.at[pl.ds(s * ZROWS, ZROWS)])
    tbase = s * TBLK
    pltpu.sync_copy(idx0_hbm.at[pl.ds(tbase, TBLK)], idx0_st)
    pltpu.sync_copy(idx1_hbm.at[c, pl.ds(tbase, TBLK)], idx1_st)
    plsc.subcore_barrier()

    def issue(b, ch):
        pltpu.async_copy(tbl_hbm.at[idx1_st.at[ch]], rows[b], gsem[b])

    def drain_scatter(b, ch):
        pltpu.make_async_copy(tbl_hbm.at[idx1_st.at[ch]], rows[b],
                              gsem[b]).wait()
        pltpu.sync_copy(rows[b], acc_sh.at[idx0_st.at[ch]], add=True)

    for b in range(NBUF):
        issue(b, b)

    def rnd(i, carry):
        for b in range(NBUF):
            ch = i * NBUF + b
            drain_scatter(b, ch)

            @pl.when(ch + NBUF < TBLK)
            def _issue(b=b, ch=ch):
                issue(b, ch + NBUF)
        return carry

    lax.fori_loop(0, NROUND, rnd, 0)
    plsc.subcore_barrier()

    r0 = s * ZROWS
    pltpu.sync_copy(acc_sh.at[pl.ds(r0, ZROWS)], acc_hbm.at[c, pl.ds(r0, ZROWS)])


@functools.partial(jax.jit, static_argnames=())
def _sc_call(idx0, idx1b, tbl, z):
    mesh = plsc.VectorSubcoreMesh(core_axis_name="c", subcore_axis_name="s",
                                  num_cores=NC, num_subcores=NS)
    kfn = pl.kernel(
        _sc_body,
        out_type=jax.ShapeDtypeStruct((NC, N_PAD, TW), jnp.float32),
        mesh=mesh,
        scratch_types=[
            pltpu.VMEM((TBLK, SC_B), jnp.int32),
            pltpu.VMEM((TBLK, SC_B), jnp.int32),
            [pltpu.VMEM((SC_B, TW), jnp.float32) for _ in range(NBUF)],
            pltpu.VMEM_SHARED((N_PAD, TW), jnp.float32),
            [pltpu.SemaphoreType.DMA for _ in range(NBUF)],
        ],
        compiler_params=pltpu.CompilerParams(use_tc_tiling_on_sc=False),
    )
    return kfn(idx0, idx1b, tbl, z)


def _post_body(a0_ref, a1_ref, att0_ref,
               ln2g, ln2b, w1a, w1b, b1, w2, b2,
               ln3g, ln3b, w31, b31, w32, b32, bias, pe2,
               out_ref):
    st = _head_expand_mat()
    a0 = a0_ref[...]
    den = a0[:, VH:] @ st + 1e-12
    att1 = jnp.concatenate([a0[:, :VH], a1_ref[:, :VH]], axis=1) / den

    a0 = att0_ref[...]
    h0 = jnp.maximum(_ln(a0, ln2g[...], ln2b[...]) @ w1a[...]
                     + pe2[0:1, :] @ w1b[...] + b1[...], 0.0)
    a0p = a0 + h0 @ w2[...] + b2[...]

    h1 = jnp.maximum(_ln(att1, ln2g[...], ln2b[...]) @ w1a[...]
                     + pe2[1:2, :] @ w1b[...] + b1[...], 0.0)
    a1p = att1 + h1 @ w2[...] + b2[...]

    xx = a0p + a1p
    h3 = jnp.maximum(_ln(xx, ln3g[...], ln3b[...]) @ w31[...] + b31[...], 0.0)
    xx = xx + h3 @ w32[...] + b32[...]
    out_ref[...] = xx + bias[...]


def _post_call(a0, a1, att0, p):
    cb = lambda shape: pl.BlockSpec(shape, lambda i: (0, 0))
    return pl.pallas_call(
        _post_body,
        grid=(_POST_GRID,),
        in_specs=[
            pl.BlockSpec((_POST_BLK, TW), lambda i: (i, 0)),
            pl.BlockSpec((_POST_BLK, TW), lambda i: (i, 0)),
            cb((1, D)),
            cb((1, D)), cb((1, D)),
            cb((D, HID)), cb((D, HID)), cb((1, HID)), cb((HID, D)), cb((1, D)),
            cb((1, D)), cb((1, D)),
            cb((D, HID)), cb((1, HID)), cb((HID, D)), cb((1, D)),
            cb((1, D)), cb((2, D)),
        ],
        out_specs=pl.BlockSpec((_POST_BLK, D), lambda i: (i, 0)),
        out_shape=jax.ShapeDtypeStruct((N, D), jnp.float32),
    )(
        a0, a1, att0,
        p['ln2g'].reshape(1, D), p['ln2b'].reshape(1, D),
        p['m2W1'][:D], p['m2W1'][D:], p['m2b1'].reshape(1, HID),
        p['m2W2'], p['m2b2'].reshape(1, D),
        p['ln3g'].reshape(1, D), p['ln3b'].reshape(1, D),
        p['m3W1'], p['m3b1'].reshape(1, HID), p['m3W2'], p['m3b2'].reshape(1, D),
        p['bias'].reshape(1, D), jnp.asarray(_PE2),
    )


def kernel(x_v, x_e, edge_orders, indices_with_nodes, params):
    x = jnp.concatenate([x_e, x_v], axis=0)
    orders = jnp.concatenate(
        [edge_orders.astype(jnp.int32), jnp.ones((N,), jnp.int32)])
    oh = (orders[:, None] == jnp.arange(32, dtype=jnp.int32)[None, :]
          ).astype(jnp.float32)

    v_all, lg, att0, gmax_rep = _pre_call(x, oh, params)

    gmax_hot = jnp.concatenate(
        [gmax_rep[0, ::DH], jnp.full((8,), 1e30, jnp.float32)]).reshape(1, 16)
    t0, t1 = _mid_call(lg, v_all, gmax_hot)
    tbl = jnp.concatenate([t0, t1], axis=0)

    i0 = indices_with_nodes[0].reshape(NCHUNK, SC_B)
    i1 = indices_with_nodes[1].reshape(NCHUNK, SC_B)
    i1b = jnp.stack([i1, i1 + EN])
    z = jnp.zeros((N_PAD, TW), jnp.float32)
    acc = _sc_call(i0, i1b, tbl, z)

    return _post_call(acc[0, :N], acc[1, :N], att0, params)


# speedup vs baseline: 136.3171x; 1.2775x over previous
import functools
import math

import jax
import jax.numpy as jnp
import numpy as np
from jax import lax
from jax.experimental import pallas as pl
from jax.experimental.pallas import tpu as pltpu
from jax.experimental.pallas import tpu_sc as plsc

N = 10000
E = 5000
M = 320000
EN = E + N
D = 128
H = 8
DH = 16
HID = 256
PE_DIM = 64
MAXK = 16

NC = 2
NS = 16
SC_B = 80
NCHUNK = M // SC_B
TBLK = NCHUNK // NS
NBUF = 5
NROUND = TBLK // NBUF
TW = 80
VH = 64
N_PAD = 10112
ZROWS = N_PAD // NS


def _sin_pe(max_len, d):
    pos = np.arange(max_len)[:, None].astype(np.float64)
    div = np.exp(np.arange(0, d, 2).astype(np.float64) * (-math.log(10000.0) / d))
    pe = np.zeros((max_len, d), dtype=np.float64)
    pe[:, 0::2] = np.sin(pos * div)
    pe[:, 1::2] = np.cos(pos * div)
    return pe.astype(np.float32)


_PE1 = np.zeros((32, D), np.float32)
_PE1[: MAXK + 1] = _sin_pe(MAXK + 1, D)
_PE2 = _sin_pe(2, D)
_PEQ = _sin_pe(2, PE_DIM)

_PRE_BLK = 600
_PRE_GRID = EN // _PRE_BLK
_POST_BLK = 1000
_POST_GRID = N // _POST_BLK


def _ln(x, g, b):
    mu = jnp.mean(x, axis=-1, keepdims=True)
    var = jnp.mean((x - mu) ** 2, axis=-1, keepdims=True)
    return (x - mu) * lax.rsqrt(var + 1e-5) * g + b


def _head_expand_mat(dtype=jnp.float32):
    r = lax.broadcasted_iota(jnp.int32, (16, D), 0)
    c = lax.broadcasted_iota(jnp.int32, (16, D), 1) // DH
    return (r == c).astype(dtype)


def _pre_body(x_ref, ord_ref, ln1g, ln1b, pe1t, w1a, w1b, b1, w2, b2,
              kw, kb, vw, vb, qw1, qb1, qw2, qb2, peq,
              v_ref, lg_ref, att0_ref, gmax_ref,
              macc, sacc, aacc, gmacc):
    i = pl.program_id(0)

    xb = x_ref[...]
    ln = _ln(xb, ln1g[...], ln1b[...])
    oh = (ord_ref[...] == lax.broadcasted_iota(
        jnp.int32, (_PRE_BLK, 32), 1)).astype(jnp.float32)
    pe = oh @ pe1t[...]
    h1 = jnp.maximum(ln @ w1a[...] + pe @ w1b[...] + b1[...], 0.0)
    xh = xb + h1 @ w2[...] + b2[...]
    k = xh @ kw[...] + kb[...]
    vv = xh @ vw[...] + vb[...]
    v_ref[...] = vv

    qh = jnp.maximum(peq[...] @ qw1[...] + qb1[...], 0.0) @ qw2[...] + qb2[...]
    q0f = qh[0:1, :]
    q1f = qh[1:2, :]

    sr = lax.broadcasted_iota(jnp.int32, (D, 16), 0) // DH
    sc = lax.broadcasted_iota(jnp.int32, (D, 16), 1)
    s16 = (sr == sc).astype(jnp.float32)
    st = _head_expand_mat()

    l0s = ((k[:, :D] * q0f) @ s16) * 0.25
    l1s = ((k[:, D:] * q1f) @ s16) * 0.25
    lg_ref[...] = l1s
    l0e = l0s @ st
    l1e = l1s @ st

    @pl.when(i == 0)
    def _init():
        macc[...] = jnp.full((1, D), -1e30, jnp.float32)
        sacc[...] = jnp.zeros((1, D), jnp.float32)
        aacc[...] = jnp.zeros((1, D), jnp.float32)
        gmacc[...] = jnp.full((1, D), -1e30, jnp.float32)

    m_old = macc[...]
    m_new = jnp.maximum(m_old, jnp.max(l0e, axis=0, keepdims=True))
    corr = jnp.exp(m_old - m_new)
    p = jnp.exp(l0e - m_new)
    sacc[...] = sacc[...] * corr + jnp.sum(p, axis=0, keepdims=True)
    aacc[...] = aacc[...] * corr + jnp.sum(p * vv, axis=0, keepdims=True)
    macc[...] = m_new
    gmacc[...] = jnp.maximum(gmacc[...], jnp.max(l1e, axis=0, keepdims=True))

    @pl.when(i == _PRE_GRID - 1)
    def _fin():
        att0_ref[...] = aacc[...] / sacc[...]
        gmax_ref[...] = gmacc[...]


def _pre_call(x, oh, p):
    cb = lambda shape: pl.BlockSpec(shape, lambda i: (0, 0))
    out_shapes = (
        jax.ShapeDtypeStruct((EN, D), jnp.float32),
        jax.ShapeDtypeStruct((EN, 16), jnp.float32),
        jax.ShapeDtypeStruct((1, D), jnp.float32),
        jax.ShapeDtypeStruct((1, D), jnp.float32),
    )
    return pl.pallas_call(
        _pre_body,
        grid=(_PRE_GRID,),
        in_specs=[
            pl.BlockSpec((_PRE_BLK, D), lambda i: (i, 0)),
            pl.BlockSpec((_PRE_BLK, 1), lambda i: (i, 0)),
            cb((1, D)), cb((1, D)), cb((32, D)),
            cb((D, HID)), cb((D, HID)), cb((1, HID)), cb((HID, D)), cb((1, D)),
            cb((D, 2 * D)), cb((1, 2 * D)), cb((D, D)), cb((1, D)),
            cb((PE_DIM, HID)), cb((1, HID)), cb((HID, H * DH)), cb((1, H * DH)),
            cb((2, PE_DIM)),
        ],
        out_specs=(
            pl.BlockSpec((_PRE_BLK, D), lambda i: (i, 0)),
            pl.BlockSpec((_PRE_BLK, 16), lambda i: (i, 0)),
            pl.BlockSpec((1, D), lambda i: (0, 0)),
            pl.BlockSpec((1, D), lambda i: (0, 0)),
        ),
        out_shape=out_shapes,
        scratch_shapes=[pltpu.VMEM((1, D), jnp.float32)] * 4,
    )(
        x, oh.reshape(EN, 1),
        p['ln1g'].reshape(1, D), p['ln1b'].reshape(1, D), jnp.asarray(_PE1),
        p['m1W1'][:D], p['m1W1'][D:], p['m1b1'].reshape(1, HID),
        p['m1W2'], p['m1b2'].reshape(1, D),
        p['kW'], p['kb'].reshape(1, 2 * D), p['vW'], p['vb'].reshape(1, D),
        p['qW1'], p['qb1'].reshape(1, HID), p['qW2'], p['qb2'].reshape(1, H * DH),
        jnp.asarray(_PEQ),
    )


def _mid_body(lg_ref, v_ref, gmax_ref, t0_ref, t1_ref):
    ex = jnp.exp(lg_ref[...] - gmax_ref[...])
    wv = (ex @ _head_expand_mat()) * v_ref[...]
    t0_ref[...] = jnp.concatenate([wv[:, :VH], ex], axis=1)
    t1_ref[...] = jnp.concatenate([wv[:, VH:], ex], axis=1)


def _mid_call(lg, v, gmax_hot):
    return pl.pallas_call(
        _mid_body,
        grid=(_PRE_GRID,),
        in_specs=[
            pl.BlockSpec((_PRE_BLK, 16), lambda i: (i, 0)),
            pl.BlockSpec((_PRE_BLK, D), lambda i: (i, 0)),
            pl.BlockSpec((1, 16), lambda i: (0, 0)),
        ],
        out_specs=(
            pl.BlockSpec((_PRE_BLK, TW), lambda i: (i, 0)),
            pl.BlockSpec((_PRE_BLK, TW), lambda i: (i, 0)),
        ),
        out_shape=(
            jax.ShapeDtypeStruct((EN, TW), jnp.float32),
            jax.ShapeDtypeStruct((EN, TW), jnp.float32),
        ),
    )(lg, v, gmax_hot)


def _sc_body(idx0_hbm, idx1_hbm, t0_hbm, t1_hbm, z_hbm, acc_hbm,
             idx0_st, idx1_st, rows, acc_sh, gsem):
    c = lax.axis_index("c")
    s = lax.axis_index("s")

    pltpu.sync_copy(z_hbm.at[pl.ds(s * ZROWS, ZROWS)],
                    acc_sh.at[pl.ds(s * ZROWS, ZROWS)])
    tbase = s * TBLK
    pltpu.sync_copy(idx0_hbm.at[pl.ds(tbase, TBLK)], idx0_st)
    pltpu.sync_copy(idx1_hbm.at[pl.ds(tbase, TBLK)], idx1_st)
    plsc.subcore_barrier()

    def pipeline(tbl_hbm):
        def issue(b, ch):
            pltpu.async_copy(tbl_hbm.at[idx1_st.at[ch]], rows[b], gsem[b])

        def drain_scatter(b, ch):
            pltpu.make_async_copy(tbl_hbm.at[idx1_st.at[ch]], rows[b],
                                  gsem[b]).wait()
            pltpu.sync_copy(rows[b], acc_sh.at[idx0_st.at[ch]], add=True)

        for b in range(NBUF):
            issue(b, b)

        def rnd(i, carry):
            for b in range(NBUF):
                ch = i * NBUF + b
                drain_scatter(b, ch)

                @pl.when(ch + NBUF < TBLK)
                def _issue(b=b, ch=ch):
                    issue(b, ch + NBUF)
            return carry

        lax.fori_loop(0, NROUND, rnd, 0)

    @pl.when(c == 0)
    def _core0():
        pipeline(t0_hbm)

    @pl.when(c == 1)
    def _core1():
        pipeline(t1_hbm)

    plsc.subcore_barrier()

    r0 = s * ZROWS
    pltpu.sync_copy(acc_sh.at[pl.ds(r0, ZROWS)], acc_hbm.at[c, pl.ds(r0, ZROWS)])


@functools.partial(jax.jit, static_argnames=())
def _sc_call(idx0, idx1, t0, t1, z):
    mesh = plsc.VectorSubcoreMesh(core_axis_name="c", subcore_axis_name="s",
                                  num_cores=NC, num_subcores=NS)
    kfn = pl.kernel(
        _sc_body,
        out_type=jax.ShapeDtypeStruct((NC, N_PAD, TW), jnp.float32),
        mesh=mesh,
        scratch_types=[
            pltpu.VMEM((TBLK, SC_B), jnp.int32),
            pltpu.VMEM((TBLK, SC_B), jnp.int32),
            [pltpu.VMEM((SC_B, TW), jnp.float32) for _ in range(NBUF)],
            pltpu.VMEM_SHARED((N_PAD, TW), jnp.float32),
            [pltpu.SemaphoreType.DMA for _ in range(NBUF)],
        ],
        compiler_params=pltpu.CompilerParams(use_tc_tiling_on_sc=False),
    )
    return kfn(idx0, idx1, t0, t1, z)


def _post_body(a0_ref, a1_ref, att0_ref,
               ln2g, ln2b, w1a, w1b, b1, w2, b2,
               ln3g, ln3b, w31, b31, w32, b32, bias, pe2,
               out_ref):
    st = _head_expand_mat()
    a0 = a0_ref[...]
    den = a0[:, VH:] @ st + 1e-12
    att1 = jnp.concatenate([a0[:, :VH], a1_ref[:, :VH]], axis=1) / den

    a0 = att0_ref[...]
    h0 = jnp.maximum(_ln(a0, ln2g[...], ln2b[...]) @ w1a[...]
                     + pe2[0:1, :] @ w1b[...] + b1[...], 0.0)
    a0p = a0 + h0 @ w2[...] + b2[...]

    h1 = jnp.maximum(_ln(att1, ln2g[...], ln2b[...]) @ w1a[...]
                     + pe2[1:2, :] @ w1b[...] + b1[...], 0.0)
    a1p = att1 + h1 @ w2[...] + b2[...]

    xx = a0p + a1p
    h3 = jnp.maximum(_ln(xx, ln3g[...], ln3b[...]) @ w31[...] + b31[...], 0.0)
    xx = xx + h3 @ w32[...] + b32[...]
    out_ref[...] = xx + bias[...]


def _post_call(a0, a1, att0, p):
    cb = lambda shape: pl.BlockSpec(shape, lambda i: (0, 0))
    return pl.pallas_call(
        _post_body,
        grid=(_POST_GRID,),
        in_specs=[
            pl.BlockSpec((_POST_BLK, TW), lambda i: (i, 0)),
            pl.BlockSpec((_POST_BLK, TW), lambda i: (i, 0)),
            cb((1, D)),
            cb((1, D)), cb((1, D)),
            cb((D, HID)), cb((D, HID)), cb((1, HID)), cb((HID, D)), cb((1, D)),
            cb((1, D)), cb((1, D)),
            cb((D, HID)), cb((1, HID)), cb((HID, D)), cb((1, D)),
            cb((1, D)), cb((2, D)),
        ],
        out_specs=pl.BlockSpec((_POST_BLK, D), lambda i: (i, 0)),
        out_shape=jax.ShapeDtypeStruct((N, D), jnp.float32),
    )(
        a0, a1, att0,
        p['ln2g'].reshape(1, D), p['ln2b'].reshape(1, D),
        p['m2W1'][:D], p['m2W1'][D:], p['m2b1'].reshape(1, HID),
        p['m2W2'], p['m2b2'].reshape(1, D),
        p['ln3g'].reshape(1, D), p['ln3b'].reshape(1, D),
        p['m3W1'], p['m3b1'].reshape(1, HID), p['m3W2'], p['m3b2'].reshape(1, D),
        p['bias'].reshape(1, D), jnp.asarray(_PE2),
    )


def kernel(x_v, x_e, edge_orders, indices_with_nodes, params):
    x = jnp.concatenate([x_e, x_v], axis=0)
    oh = jnp.concatenate(
        [edge_orders.astype(jnp.int32), jnp.ones((N,), jnp.int32)])

    v_all, lg, att0, gmax_rep = _pre_call(x, oh, params)

    gmax_hot = jnp.concatenate(
        [gmax_rep[0, ::DH], jnp.full((8,), 1e30, jnp.float32)]).reshape(1, 16)
    t0, t1 = _mid_call(lg, v_all, gmax_hot)

    i0 = indices_with_nodes[0].reshape(NCHUNK, SC_B)
    i1 = indices_with_nodes[1].reshape(NCHUNK, SC_B)
    z = jnp.zeros((N_PAD, TW), jnp.float32)
    acc = _sc_call(i0, i1, t0, t1, z)

    return _post_call(acc[0], acc[1], att0, params)


# speedup vs baseline: 149.2981x; 1.0952x over previous
import functools
import math

import jax
import jax.numpy as jnp
import numpy as np
from jax import lax
from jax.experimental import pallas as pl
from jax.experimental.pallas import tpu as pltpu
from jax.experimental.pallas import tpu_sc as plsc

N = 10000
E = 5000
M = 320000
EN = E + N
D = 128
H = 8
DH = 16
HID = 256
PE_DIM = 64
MAXK = 16

NC = 2
NS = 16
SC_B = 80
NCHUNK = M // SC_B
TBLK = NCHUNK // NS
NBUF = 5
NROUND = TBLK // NBUF
TW = 80
VH = 64
N_PAD = 10112
ZROWS = N_PAD // NS


def _sin_pe(max_len, d):
    pos = np.arange(max_len)[:, None].astype(np.float64)
    div = np.exp(np.arange(0, d, 2).astype(np.float64) * (-math.log(10000.0) / d))
    pe = np.zeros((max_len, d), dtype=np.float64)
    pe[:, 0::2] = np.sin(pos * div)
    pe[:, 1::2] = np.cos(pos * div)
    return pe.astype(np.float32)


_PE1 = np.zeros((32, D), np.float32)
_PE1[: MAXK + 1] = _sin_pe(MAXK + 1, D)
_PE2 = _sin_pe(2, D)
_PEQ = _sin_pe(2, PE_DIM)

_PRE_BLK = 1000
_PRE_GRID = EN // _PRE_BLK
_EBLK = E // _PRE_BLK
_POST_BLK = 1000
_POST_GRID = N // _POST_BLK


def _ln(x, g, b):
    mu = jnp.mean(x, axis=-1, keepdims=True)
    var = jnp.mean((x - mu) ** 2, axis=-1, keepdims=True)
    return (x - mu) * lax.rsqrt(var + 1e-5) * g + b


def _head_expand_mat(dtype=jnp.float32):
    r = lax.broadcasted_iota(jnp.int32, (16, D), 0)
    c = lax.broadcasted_iota(jnp.int32, (16, D), 1) // DH
    return (r == c).astype(dtype)


def _pre_body(xe_ref, xv_ref, ord_ref, ln1g, ln1b, pe1t, w1a, w1b, b1, w2, b2,
              kw, kb, vw, vb, qw1, qb1, qw2, qb2, peq,
              v_ref, lg_ref, att0_ref, gmax_ref,
              macc, sacc, aacc, gmacc):
    i = pl.program_id(0)

    is_edge = i < _EBLK
    xb = jnp.where(is_edge, xe_ref[...], xv_ref[...])
    ordb = jnp.where(is_edge, ord_ref[...], 1)
    ln = _ln(xb, ln1g[...], ln1b[...])
    oh = (ordb == lax.broadcasted_iota(
        jnp.int32, (_PRE_BLK, 32), 1)).astype(jnp.float32)
    pe = oh @ pe1t[...]
    h1 = jnp.maximum(ln @ w1a[...] + pe @ w1b[...] + b1[...], 0.0)
    xh = xb + h1 @ w2[...] + b2[...]
    k = xh @ kw[...] + kb[...]
    vv = xh @ vw[...] + vb[...]
    v_ref[...] = vv

    qh = jnp.maximum(peq[...] @ qw1[...] + qb1[...], 0.0) @ qw2[...] + qb2[...]
    q0f = qh[0:1, :]
    q1f = qh[1:2, :]

    sr = lax.broadcasted_iota(jnp.int32, (D, 16), 0) // DH
    sc = lax.broadcasted_iota(jnp.int32, (D, 16), 1)
    s16 = (sr == sc).astype(jnp.float32)
    st = _head_expand_mat()

    l0s = ((k[:, :D] * q0f) @ s16) * 0.25
    l1s = ((k[:, D:] * q1f) @ s16) * 0.25
    lg_ref[...] = l1s
    l0e = l0s @ st
    l1e = l1s @ st

    @pl.when(i == 0)
    def _init():
        macc[...] = jnp.full((1, D), -1e30, jnp.float32)
        sacc[...] = jnp.zeros((1, D), jnp.float32)
        aacc[...] = jnp.zeros((1, D), jnp.float32)
        gmacc[...] = jnp.full((1, D), -1e30, jnp.float32)

    m_old = macc[...]
    m_new = jnp.maximum(m_old, jnp.max(l0e, axis=0, keepdims=True))
    corr = jnp.exp(m_old - m_new)
    p = jnp.exp(l0e - m_new)
    sacc[...] = sacc[...] * corr + jnp.sum(p, axis=0, keepdims=True)
    aacc[...] = aacc[...] * corr + jnp.sum(p * vv, axis=0, keepdims=True)
    macc[...] = m_new
    gmacc[...] = jnp.maximum(gmacc[...], jnp.max(l1e, axis=0, keepdims=True))

    @pl.when(i == _PRE_GRID - 1)
    def _fin():
        att0_ref[...] = aacc[...] / sacc[...]
        gmax_ref[...] = gmacc[...]


def _pre_call(x_e, x_v, orde, p):
    cb = lambda shape: pl.BlockSpec(shape, lambda i: (0, 0))
    out_shapes = (
        jax.ShapeDtypeStruct((EN, D), jnp.float32),
        jax.ShapeDtypeStruct((EN, 16), jnp.float32),
        jax.ShapeDtypeStruct((1, D), jnp.float32),
        jax.ShapeDtypeStruct((1, D), jnp.float32),
    )
    return pl.pallas_call(
        _pre_body,
        grid=(_PRE_GRID,),
        in_specs=[
            pl.BlockSpec((_PRE_BLK, D),
                         lambda i: (jnp.minimum(i, _EBLK - 1), 0)),
            pl.BlockSpec((_PRE_BLK, D),
                         lambda i: (jnp.maximum(i - _EBLK, 0), 0)),
            pl.BlockSpec((_PRE_BLK, 1),
                         lambda i: (jnp.minimum(i, _EBLK - 1), 0)),
            cb((1, D)), cb((1, D)), cb((32, D)),
            cb((D, HID)), cb((D, HID)), cb((1, HID)), cb((HID, D)), cb((1, D)),
            cb((D, 2 * D)), cb((1, 2 * D)), cb((D, D)), cb((1, D)),
            cb((PE_DIM, HID)), cb((1, HID)), cb((HID, H * DH)), cb((1, H * DH)),
            cb((2, PE_DIM)),
        ],
        out_specs=(
            pl.BlockSpec((_PRE_BLK, D), lambda i: (i, 0)),
            pl.BlockSpec((_PRE_BLK, 16), lambda i: (i, 0)),
            pl.BlockSpec((1, D), lambda i: (0, 0)),
            pl.BlockSpec((1, D), lambda i: (0, 0)),
        ),
        out_shape=out_shapes,
        scratch_shapes=[pltpu.VMEM((1, D), jnp.float32)] * 4,
    )(
        x_e, x_v, orde.reshape(E, 1),
        p['ln1g'].reshape(1, D), p['ln1b'].reshape(1, D), jnp.asarray(_PE1),
        p['m1W1'][:D], p['m1W1'][D:], p['m1b1'].reshape(1, HID),
        p['m1W2'], p['m1b2'].reshape(1, D),
        p['kW'], p['kb'].reshape(1, 2 * D), p['vW'], p['vb'].reshape(1, D),
        p['qW1'], p['qb1'].reshape(1, HID), p['qW2'], p['qb2'].reshape(1, H * DH),
        jnp.asarray(_PEQ),
    )


def _mid_body(lg_ref, v_ref, gmax_ref, t0_ref, t1_ref):
    ex = jnp.exp(lg_ref[...] - gmax_ref[...])
    wv = (ex @ _head_expand_mat()) * v_ref[...]
    t0_ref[...] = jnp.concatenate([wv[:, :VH], ex], axis=1)
    t1_ref[...] = jnp.concatenate([wv[:, VH:], ex], axis=1)


def _mid_call(lg, v, gmax_hot):
    return pl.pallas_call(
        _mid_body,
        grid=(_PRE_GRID,),
        in_specs=[
            pl.BlockSpec((_PRE_BLK, 16), lambda i: (i, 0)),
            pl.BlockSpec((_PRE_BLK, D), lambda i: (i, 0)),
            pl.BlockSpec((1, 16), lambda i: (0, 0)),
        ],
        out_specs=(
            pl.BlockSpec((_PRE_BLK, TW), lambda i: (i, 0)),
            pl.BlockSpec((_PRE_BLK, TW), lambda i: (i, 0)),
        ),
        out_shape=(
            jax.ShapeDtypeStruct((EN, TW), jnp.float32),
            jax.ShapeDtypeStruct((EN, TW), jnp.float32),
        ),
    )(lg, v, gmax_hot)


def _sc_body(idx0_hbm, idx1_hbm, t0_hbm, t1_hbm, z_hbm, acc_hbm,
             idx0_st, idx1_st, rows, acc_sh, gsem):
    c = lax.axis_index("c")
    s = lax.axis_index("s")

    pltpu.sync_copy(z_hbm.at[pl.ds(s * ZROWS, ZROWS)],
                    acc_sh.at[pl.ds(s * ZROWS, ZROWS)])
    tbase = s * TBLK
    pltpu.sync_copy(idx0_hbm.at[pl.ds(tbase, TBLK)], idx0_st)
    pltpu.sync_copy(idx1_hbm.at[pl.ds(tbase, TBLK)], idx1_st)
    plsc.subcore_barrier()

    def pipeline(tbl_hbm):
        def issue(b, ch):
            pltpu.async_copy(tbl_hbm.at[idx1_st.at[ch]], rows[b], gsem[b])

        def drain_scatter(b, ch):
            pltpu.make_async_copy(tbl_hbm.at[idx1_st.at[ch]], rows[b],
                                  gsem[b]).wait()
            pltpu.sync_copy(rows[b], acc_sh.at[idx0_st.at[ch]], add=True)

        for b in range(NBUF):
            issue(b, b)

        def rnd(i, carry):
            for b in range(NBUF):
                ch = i * NBUF + b
                drain_scatter(b, ch)

                @pl.when(ch + NBUF < TBLK)
                def _issue(b=b, ch=ch):
                    issue(b, ch + NBUF)
            return carry

        lax.fori_loop(0, NROUND, rnd, 0)

    @pl.when(c == 0)
    def _core0():
        pipeline(t0_hbm)

    @pl.when(c == 1)
    def _core1():
        pipeline(t1_hbm)

    plsc.subcore_barrier()

    r0 = s * ZROWS
    pltpu.sync_copy(acc_sh.at[pl.ds(r0, ZROWS)], acc_hbm.at[c, pl.ds(r0, ZROWS)])


@functools.partial(jax.jit, static_argnames=())
def _sc_call(idx0, idx1, t0, t1, z):
    mesh = plsc.VectorSubcoreMesh(core_axis_name="c", subcore_axis_name="s",
                                  num_cores=NC, num_subcores=NS)
    kfn = pl.kernel(
        _sc_body,
        out_type=jax.ShapeDtypeStruct((NC, N_PAD, TW), jnp.float32),
        mesh=mesh,
        scratch_types=[
            pltpu.VMEM((TBLK, SC_B), jnp.int32),
            pltpu.VMEM((TBLK, SC_B), jnp.int32),
            [pltpu.VMEM((SC_B, TW), jnp.float32) for _ in range(NBUF)],
            pltpu.VMEM_SHARED((N_PAD, TW), jnp.float32),
            [pltpu.SemaphoreType.DMA for _ in range(NBUF)],
        ],
        compiler_params=pltpu.CompilerParams(use_tc_tiling_on_sc=False),
    )
    return kfn(idx0, idx1, t0, t1, z)


def _post_body(a0_ref, a1_ref, att0_ref,
               ln2g, ln2b, w1a, w1b, b1, w2, b2,
               ln3g, ln3b, w31, b31, w32, b32, bias, pe2,
               out_ref):
    st = _head_expand_mat()
    a0 = a0_ref[...]
    den = a0[:, VH:] @ st + 1e-12
    att1 = jnp.concatenate([a0[:, :VH], a1_ref[:, :VH]], axis=1) / den

    a0 = att0_ref[...]
    h0 = jnp.maximum(_ln(a0, ln2g[...], ln2b[...]) @ w1a[...]
                     + pe2[0:1, :] @ w1b[...] + b1[...], 0.0)
    a0p = a0 + h0 @ w2[...] + b2[...]

    h1 = jnp.maximum(_ln(att1, ln2g[...], ln2b[...]) @ w1a[...]
                     + pe2[1:2, :] @ w1b[...] + b1[...], 0.0)
    a1p = att1 + h1 @ w2[...] + b2[...]

    xx = a0p + a1p
    h3 = jnp.maximum(_ln(xx, ln3g[...], ln3b[...]) @ w31[...] + b31[...], 0.0)
    xx = xx + h3 @ w32[...] + b32[...]
    out_ref[...] = xx + bias[...]


def _post_call(a0, a1, att0, p):
    cb = lambda shape: pl.BlockSpec(shape, lambda i: (0, 0))
    return pl.pallas_call(
        _post_body,
        grid=(_POST_GRID,),
        in_specs=[
            pl.BlockSpec((_POST_BLK, TW), lambda i: (i, 0)),
            pl.BlockSpec((_POST_BLK, TW), lambda i: (i, 0)),
            cb((1, D)),
            cb((1, D)), cb((1, D)),
            cb((D, HID)), cb((D, HID)), cb((1, HID)), cb((HID, D)), cb((1, D)),
            cb((1, D)), cb((1, D)),
            cb((D, HID)), cb((1, HID)), cb((HID, D)), cb((1, D)),
            cb((1, D)), cb((2, D)),
        ],
        out_specs=pl.BlockSpec((_POST_BLK, D), lambda i: (i, 0)),
        out_shape=jax.ShapeDtypeStruct((N, D), jnp.float32),
    )(
        a0, a1, att0,
        p['ln2g'].reshape(1, D), p['ln2b'].reshape(1, D),
        p['m2W1'][:D], p['m2W1'][D:], p['m2b1'].reshape(1, HID),
        p['m2W2'], p['m2b2'].reshape(1, D),
        p['ln3g'].reshape(1, D), p['ln3b'].reshape(1, D),
        p['m3W1'], p['m3b1'].reshape(1, HID), p['m3W2'], p['m3b2'].reshape(1, D),
        p['bias'].reshape(1, D), jnp.asarray(_PE2),
    )


def kernel(x_v, x_e, edge_orders, indices_with_nodes, params):
    v_all, lg, att0, gmax_rep = _pre_call(
        x_e, x_v, edge_orders.astype(jnp.int32), params)

    gmax_hot = jnp.concatenate(
        [gmax_rep[0, ::DH], jnp.full((8,), 1e30, jnp.float32)]).reshape(1, 16)
    t0, t1 = _mid_call(lg, v_all, gmax_hot)

    i0 = indices_with_nodes[0].reshape(NCHUNK, SC_B)
    i1 = indices_with_nodes[1].reshape(NCHUNK, SC_B)
    z = jnp.zeros((N_PAD, TW), jnp.float32)
    acc = _sc_call(i0, i1, t0, t1, z)

    return _post_call(acc[0], acc[1], att0, params)


# speedup vs baseline: 149.6699x; 1.0025x over previous
import functools
import math

import jax
import jax.numpy as jnp
import numpy as np
from jax import lax
from jax.experimental import pallas as pl
from jax.experimental.pallas import tpu as pltpu
from jax.experimental.pallas import tpu_sc as plsc

N = 10000
E = 5000
M = 320000
EN = E + N
D = 128
H = 8
DH = 16
HID = 256
PE_DIM = 64
MAXK = 16

NC = 2
NS = 16
SC_B = 80
NCHUNK = M // SC_B
TBLK = NCHUNK // NS
NBUF = 5
NROUND = TBLK // NBUF
TW = 80
VH = 64
N_PAD = 10112
ZROWS = N_PAD // NS


def _sin_pe(max_len, d):
    pos = np.arange(max_len)[:, None].astype(np.float64)
    div = np.exp(np.arange(0, d, 2).astype(np.float64) * (-math.log(10000.0) / d))
    pe = np.zeros((max_len, d), dtype=np.float64)
    pe[:, 0::2] = np.sin(pos * div)
    pe[:, 1::2] = np.cos(pos * div)
    return pe.astype(np.float32)


_PE1 = np.zeros((32, D), np.float32)
_PE1[: MAXK + 1] = _sin_pe(MAXK + 1, D)
_PE2 = _sin_pe(2, D)
_PEQ = _sin_pe(2, PE_DIM)

_PRE_BLK = 1000
_PRE_GRID = EN // _PRE_BLK
_EBLK = E // _PRE_BLK
_POST_BLK = 1000
_POST_GRID = N // _POST_BLK


def _ln(x, g, b):
    mu = jnp.mean(x, axis=-1, keepdims=True)
    var = jnp.mean((x - mu) ** 2, axis=-1, keepdims=True)
    return (x - mu) * lax.rsqrt(var + 1e-5) * g + b


def _head_expand_mat(dtype=jnp.float32):
    r = lax.broadcasted_iota(jnp.int32, (16, D), 0)
    c = lax.broadcasted_iota(jnp.int32, (16, D), 1) // DH
    return (r == c).astype(dtype)


def _pre_body(xe_ref, xv_ref, ord_ref, ln1g, ln1b, pe1t, w1a, w1b, b1, w2, b2,
              kw, kb, vw, vb, qw1, qb1, qw2, qb2, peq,
              v_ref, lg_ref, att0_ref, gmax_ref,
              macc, sacc, aacc, gmacc):
    i = pl.program_id(0)

    is_edge = i < _EBLK
    xb = jnp.where(is_edge, xe_ref[...], xv_ref[...])
    ordb = jnp.where(is_edge, ord_ref[...], 1)
    ln = _ln(xb, ln1g[...], ln1b[...])
    oh = (ordb == lax.broadcasted_iota(
        jnp.int32, (_PRE_BLK, 32), 1)).astype(jnp.float32)
    pe = oh @ pe1t[...]
    h1 = jnp.maximum(ln @ w1a[...] + pe @ w1b[...] + b1[...], 0.0)
    xh = xb + h1 @ w2[...] + b2[...]
    k = xh @ kw[...] + kb[...]
    vv = xh @ vw[...] + vb[...]
    v_ref[...] = vv

    qh = jnp.maximum(peq[...] @ qw1[...] + qb1[...], 0.0) @ qw2[...] + qb2[...]
    q0f = qh[0:1, :]
    q1f = qh[1:2, :]

    sr = lax.broadcasted_iota(jnp.int32, (D, 16), 0) // DH
    sc = lax.broadcasted_iota(jnp.int32, (D, 16), 1)
    s16 = (sr == sc).astype(jnp.float32)
    st = _head_expand_mat()

    l0s = ((k[:, :D] * q0f) @ s16) * 0.25
    l1s = ((k[:, D:] * q1f) @ s16) * 0.25
    lg_ref[...] = l1s
    l0e = l0s @ st

    @pl.when(i == 0)
    def _init():
        macc[...] = jnp.full((1, D), -1e30, jnp.float32)
        sacc[...] = jnp.zeros((1, D), jnp.float32)
        aacc[...] = jnp.zeros((1, D), jnp.float32)
        gmacc[...] = jnp.full((1, 16), -1e30, jnp.float32)

    m_old = macc[...]
    m_new = jnp.maximum(m_old, jnp.max(l0e, axis=0, keepdims=True))
    corr = jnp.exp(m_old - m_new)
    p = jnp.exp(l0e - m_new)
    sacc[...] = sacc[...] * corr + jnp.sum(p, axis=0, keepdims=True)
    aacc[...] = aacc[...] * corr + jnp.sum(p * vv, axis=0, keepdims=True)
    macc[...] = m_new
    gmacc[...] = jnp.maximum(gmacc[...], jnp.max(l1s, axis=0, keepdims=True))

    @pl.when(i == _PRE_GRID - 1)
    def _fin():
        att0_ref[...] = aacc[...] / sacc[...]
        gmax_ref[...] = gmacc[...]


def _pre_call(x_e, x_v, orde, p):
    cb = lambda shape: pl.BlockSpec(shape, lambda i: (0, 0))
    out_shapes = (
        jax.ShapeDtypeStruct((EN, D), jnp.float32),
        jax.ShapeDtypeStruct((EN, 16), jnp.float32),
        jax.ShapeDtypeStruct((1, D), jnp.float32),
        jax.ShapeDtypeStruct((1, 16), jnp.float32),
    )
    return pl.pallas_call(
        _pre_body,
        grid=(_PRE_GRID,),
        in_specs=[
            pl.BlockSpec((_PRE_BLK, D),
                         lambda i: (jnp.minimum(i, _EBLK - 1), 0)),
            pl.BlockSpec((_PRE_BLK, D),
                         lambda i: (jnp.maximum(i - _EBLK, 0), 0)),
            pl.BlockSpec((_PRE_BLK, 1),
                         lambda i: (jnp.minimum(i, _EBLK - 1), 0)),
            cb((1, D)), cb((1, D)), cb((32, D)),
            cb((D, HID)), cb((D, HID)), cb((1, HID)), cb((HID, D)), cb((1, D)),
            cb((D, 2 * D)), cb((1, 2 * D)), cb((D, D)), cb((1, D)),
            cb((PE_DIM, HID)), cb((1, HID)), cb((HID, H * DH)), cb((1, H * DH)),
            cb((2, PE_DIM)),
        ],
        out_specs=(
            pl.BlockSpec((_PRE_BLK, D), lambda i: (i, 0)),
            pl.BlockSpec((_PRE_BLK, 16), lambda i: (i, 0)),
            pl.BlockSpec((1, D), lambda i: (0, 0)),
            pl.BlockSpec((1, 16), lambda i: (0, 0)),
        ),
        out_shape=out_shapes,
        scratch_shapes=[pltpu.VMEM((1, D), jnp.float32)] * 3
        + [pltpu.VMEM((1, 16), jnp.float32)],
    )(
        x_e, x_v, orde.reshape(E, 1),
        p['ln1g'].reshape(1, D), p['ln1b'].reshape(1, D), jnp.asarray(_PE1),
        p['m1W1'][:D], p['m1W1'][D:], p['m1b1'].reshape(1, HID),
        p['m1W2'], p['m1b2'].reshape(1, D),
        p['kW'], p['kb'].reshape(1, 2 * D), p['vW'], p['vb'].reshape(1, D),
        p['qW1'], p['qb1'].reshape(1, HID), p['qW2'], p['qb2'].reshape(1, H * DH),
        jnp.asarray(_PEQ),
    )


def _mid_body(lg_ref, v_ref, gmax_ref, t0_ref, t1_ref):
    ex = jnp.exp(lg_ref[...] - gmax_ref[...])
    wv = (ex @ _head_expand_mat()) * v_ref[...]
    t0_ref[...] = jnp.concatenate([wv[:, :VH], ex], axis=1)
    t1_ref[...] = jnp.concatenate([wv[:, VH:], ex], axis=1)


def _mid_call(lg, v, gmax_hot):
    return pl.pallas_call(
        _mid_body,
        grid=(_PRE_GRID,),
        in_specs=[
            pl.BlockSpec((_PRE_BLK, 16), lambda i: (i, 0)),
            pl.BlockSpec((_PRE_BLK, D), lambda i: (i, 0)),
            pl.BlockSpec((1, 16), lambda i: (0, 0)),
        ],
        out_specs=(
            pl.BlockSpec((_PRE_BLK, TW), lambda i: (i, 0)),
            pl.BlockSpec((_PRE_BLK, TW), lambda i: (i, 0)),
        ),
        out_shape=(
            jax.ShapeDtypeStruct((EN, TW), jnp.float32),
            jax.ShapeDtypeStruct((EN, TW), jnp.float32),
        ),
    )(lg, v, gmax_hot)


def _sc_body(idx0_hbm, idx1_hbm, t0_hbm, t1_hbm, z_hbm, acc_hbm,
             idx0_st, idx1_st, rows, acc_sh, gsem):
    c = lax.axis_index("c")
    s = lax.axis_index("s")

    pltpu.sync_copy(z_hbm.at[pl.ds(s * ZROWS, ZROWS)],
                    acc_sh.at[pl.ds(s * ZROWS, ZROWS)])
    tbase = s * TBLK
    pltpu.sync_copy(idx0_hbm.at[pl.ds(tbase, TBLK)], idx0_st)
    pltpu.sync_copy(idx1_hbm.at[pl.ds(tbase, TBLK)], idx1_st)
    plsc.subcore_barrier()

    def pipeline(tbl_hbm):
        def issue(b, ch):
            pltpu.async_copy(tbl_hbm.at[idx1_st.at[ch]], rows[b], gsem[b])

        def drain_scatter(b, ch):
            pltpu.make_async_copy(tbl_hbm.at[idx1_st.at[ch]], rows[b],
                                  gsem[b]).wait()
            pltpu.sync_copy(rows[b], acc_sh.at[idx0_st.at[ch]], add=True)

        for b in range(NBUF):
            issue(b, b)

        def rnd(i, carry):
            for b in range(NBUF):
                ch = i * NBUF + b
                drain_scatter(b, ch)

                @pl.when(ch + NBUF < TBLK)
                def _issue(b=b, ch=ch):
                    issue(b, ch + NBUF)
            return carry

        lax.fori_loop(0, NROUND, rnd, 0)

    @pl.when(c == 0)
    def _core0():
        pipeline(t0_hbm)

    @pl.when(c == 1)
    def _core1():
        pipeline(t1_hbm)

    plsc.subcore_barrier()

    r0 = s * ZROWS
    pltpu.sync_copy(acc_sh.at[pl.ds(r0, ZROWS)], acc_hbm.at[c, pl.ds(r0, ZROWS)])


@functools.partial(jax.jit, static_argnames=())
def _sc_call(idx0, idx1, t0, t1, z):
    mesh = plsc.VectorSubcoreMesh(core_axis_name="c", subcore_axis_name="s",
                                  num_cores=NC, num_subcores=NS)
    kfn = pl.kernel(
        _sc_body,
        out_type=jax.ShapeDtypeStruct((NC, N_PAD, TW), jnp.float32),
        mesh=mesh,
        scratch_types=[
            pltpu.VMEM((TBLK, SC_B), jnp.int32),
            pltpu.VMEM((TBLK, SC_B), jnp.int32),
            [pltpu.VMEM((SC_B, TW), jnp.float32) for _ in range(NBUF)],
            pltpu.VMEM_SHARED((N_PAD, TW), jnp.float32),
            [pltpu.SemaphoreType.DMA for _ in range(NBUF)],
        ],
        compiler_params=pltpu.CompilerParams(use_tc_tiling_on_sc=False),
    )
    return kfn(idx0, idx1, t0, t1, z)


def _post_body(a0_ref, a1_ref, att0_ref,
               ln2g, ln2b, w1a, w1b, b1, w2, b2,
               ln3g, ln3b, w31, b31, w32, b32, bias, pe2,
               out_ref):
    st = _head_expand_mat()
    a0 = a0_ref[...]
    den = a0[:, VH:] @ st + 1e-12
    att1 = jnp.concatenate([a0[:, :VH], a1_ref[:, :VH]], axis=1) / den

    a0 = att0_ref[...]
    h0 = jnp.maximum(_ln(a0, ln2g[...], ln2b[...]) @ w1a[...]
                     + pe2[0:1, :] @ w1b[...] + b1[...], 0.0)
    a0p = a0 + h0 @ w2[...] + b2[...]

    h1 = jnp.maximum(_ln(att1, ln2g[...], ln2b[...]) @ w1a[...]
                     + pe2[1:2, :] @ w1b[...] + b1[...], 0.0)
    a1p = att1 + h1 @ w2[...] + b2[...]

    xx = a0p + a1p
    h3 = jnp.maximum(_ln(xx, ln3g[...], ln3b[...]) @ w31[...] + b31[...], 0.0)
    xx = xx + h3 @ w32[...] + b32[...]
    out_ref[...] = xx + bias[...]


def _post_call(a0, a1, att0, p):
    cb = lambda shape: pl.BlockSpec(shape, lambda i: (0, 0))
    return pl.pallas_call(
        _post_body,
        grid=(_POST_GRID,),
        in_specs=[
            pl.BlockSpec((_POST_BLK, TW), lambda i: (i, 0)),
            pl.BlockSpec((_POST_BLK, TW), lambda i: (i, 0)),
            cb((1, D)),
            cb((1, D)), cb((1, D)),
            cb((D, HID)), cb((D, HID)), cb((1, HID)), cb((HID, D)), cb((1, D)),
            cb((1, D)), cb((1, D)),
            cb((D, HID)), cb((1, HID)), cb((HID, D)), cb((1, D)),
            cb((1, D)), cb((2, D)),
        ],
        out_specs=pl.BlockSpec((_POST_BLK, D), lambda i: (i, 0)),
        out_shape=jax.ShapeDtypeStruct((N, D), jnp.float32),
    )(
        a0, a1, att0,
        p['ln2g'].reshape(1, D), p['ln2b'].reshape(1, D),
        p['m2W1'][:D], p['m2W1'][D:], p['m2b1'].reshape(1, HID),
        p['m2W2'], p['m2b2'].reshape(1, D),
        p['ln3g'].reshape(1, D), p['ln3b'].reshape(1, D),
        p['m3W1'], p['m3b1'].reshape(1, HID), p['m3W2'], p['m3b2'].reshape(1, D),
        p['bias'].reshape(1, D), jnp.asarray(_PE2),
    )


def kernel(x_v, x_e, edge_orders, indices_with_nodes, params):
    v_all, lg, att0, gmax_rep = _pre_call(
        x_e, x_v, edge_orders.astype(jnp.int32), params)

    gmax_hot = jnp.concatenate(
        [gmax_rep[0, :H], jnp.full((8,), 1e30, jnp.float32)]).reshape(1, 16)
    t0, t1 = _mid_call(lg, v_all, gmax_hot)

    i0 = indices_with_nodes[0].reshape(NCHUNK, SC_B)
    i1 = indices_with_nodes[1].reshape(NCHUNK, SC_B)
    z = jnp.zeros((N_PAD, TW), jnp.float32)
    acc = _sc_call(i0, i1, t0, t1, z)

    return _post_call(acc[0], acc[1], att0, params)
